# baseline (speedup 1.0000x reference)
import functools

import jax
import jax.numpy as jnp
from jax.experimental import pallas as pl
from jax.experimental.pallas import tpu as pltpu

_ROWS = 128
_RING = 32


def _dma_body(batch, n_chunks, w_hbm, o_hbm, *rest):
    bufs = rest[:_RING]
    rsems = rest[_RING:2 * _RING]
    wsems = rest[2 * _RING:3 * _RING]

    def read(k):
        pltpu.make_async_copy(
            w_hbm.at[pl.ds(k * _ROWS, _ROWS), :], bufs[k % _RING], rsems[k % _RING]
        ).start()

    def write_start(k):
        for b in range(batch):
            pltpu.make_async_copy(
                bufs[k % _RING], o_hbm.at[b, pl.ds(k * _ROWS, _ROWS), :],
                wsems[k % _RING],
            ).start()

    def write_wait(k):
        for b in range(batch):
            pltpu.make_async_copy(
                bufs[k % _RING], o_hbm.at[b, pl.ds(k * _ROWS, _ROWS), :],
                wsems[k % _RING],
            ).wait()

    for k in range(_RING):
        read(k)
    for k in range(n_chunks):
        pltpu.make_async_copy(
            w_hbm.at[pl.ds(k * _ROWS, _ROWS), :], bufs[k % _RING], rsems[k % _RING]
        ).wait()
        write_start(k)
        p = k - 1
        if p >= 0 and p + _RING < n_chunks:
            write_wait(p)
            read(p + _RING)
    for p in range(max(0, n_chunks - _RING - 1), n_chunks):
        if p + _RING >= n_chunks:
            write_wait(p)


def kernel(tokens, W_pos):
    B, S = tokens.shape
    D = W_pos.shape[1]
    n_chunks = S // _ROWS

    return pl.pallas_call(
        functools.partial(_dma_body, B, n_chunks),
        in_specs=[pl.BlockSpec(memory_space=pl.ANY)],
        out_specs=pl.BlockSpec(memory_space=pl.ANY),
        out_shape=jax.ShapeDtypeStruct((B, S, D), jnp.float32),
        scratch_shapes=(
            [pltpu.VMEM((_ROWS, D), jnp.float32)] * _RING
            + [pltpu.SemaphoreType.DMA] * (2 * _RING)
        ),
    )(W_pos)


# speedup vs baseline: 1.0013x; 1.0013x over previous
import functools

import jax
import jax.numpy as jnp
from jax.experimental import pallas as pl
from jax.experimental.pallas import tpu as pltpu

_ROWS = 256
_RING = 16


def _dma_body(batch, n_chunks, w_hbm, o_hbm, *rest):
    bufs = rest[:_RING]
    rsems = rest[_RING:2 * _RING]
    wsems = rest[2 * _RING:3 * _RING]

    def read(k):
        pltpu.make_async_copy(
            w_hbm.at[pl.ds(k * _ROWS, _ROWS), :], bufs[k % _RING], rsems[k % _RING]
        ).start()

    def write_start(k):
        for b in range(batch):
            pltpu.make_async_copy(
                bufs[k % _RING], o_hbm.at[b, pl.ds(k * _ROWS, _ROWS), :],
                wsems[k % _RING],
            ).start()

    def write_wait(k):
        for b in range(batch):
            pltpu.make_async_copy(
                bufs[k % _RING], o_hbm.at[b, pl.ds(k * _ROWS, _ROWS), :],
                wsems[k % _RING],
            ).wait()

    for k in range(_RING):
        read(k)
    for k in range(n_chunks):
        pltpu.make_async_copy(
            w_hbm.at[pl.ds(k * _ROWS, _ROWS), :], bufs[k % _RING], rsems[k % _RING]
        ).wait()
        write_start(k)
        p = k - 1
        if p >= 0 and p + _RING < n_chunks:
            write_wait(p)
            read(p + _RING)
    for p in range(max(0, n_chunks - _RING - 1), n_chunks):
        if p + _RING >= n_chunks:
            write_wait(p)


def kernel(tokens, W_pos):
    B, S = tokens.shape
    D = W_pos.shape[1]
    n_chunks = S // _ROWS

    return pl.pallas_call(
        functools.partial(_dma_body, B, n_chunks),
        in_specs=[pl.BlockSpec(memory_space=pl.ANY)],
        out_specs=pl.BlockSpec(memory_space=pl.ANY),
        out_shape=jax.ShapeDtypeStruct((B, S, D), jnp.float32),
        scratch_shapes=(
            [pltpu.VMEM((_ROWS, D), jnp.float32)] * _RING
            + [pltpu.SemaphoreType.DMA] * (2 * _RING)
        ),
    )(W_pos)
